# baseline (speedup 1.0000x reference)
import functools

import jax
import jax.numpy as jnp
from jax import lax
from jax.experimental import pallas as pl
from jax.experimental.pallas import tpu as pltpu
from jax.experimental.pallas import tpu_sc as plsc

_NC = 2
_NS = 16
_CHUNK = 128
_BM = 512

_CPW0 = 56
_CPW1 = 104
_WMAX = 56


def _matmul_body(xe_ref, w_ref, o_ref):
    o_ref[...] = jnp.dot(xe_ref[...], w_ref[...],
                         preferred_element_type=jnp.float32)


def _combine_body(p0_ref, p1_ref, o_ref):
    o_ref[...] = jnp.maximum(p0_ref[...] + p1_ref[...], 0.0)


def _sc_segment_sum(table, gidx, sidx, zeros, D, H):
    mesh = plsc.VectorSubcoreMesh(core_axis_name="c", subcore_axis_name="s")
    out_rows = H // _NS
    zero_rows = H // _NS

    @functools.partial(
        pl.kernel,
        out_type=jax.ShapeDtypeStruct((_NC, H, D), jnp.float32),
        mesh=mesh,
        scratch_types=[
            pltpu.VMEM((max(_CPW0, _CPW1), _CHUNK), jnp.int32),
            pltpu.VMEM((max(_CPW0, _CPW1), _CHUNK), jnp.int32),
            pltpu.VMEM((_CHUNK, D), jnp.float32),
            pltpu.VMEM_SHARED((H, D), jnp.float32),
        ],
    )
    def k(table_hbm, gidx_hbm, sidx_hbm, zeros_hbm, out_hbm,
          gidx_v, sidx_v, rows_v, h_sh):
        cid = lax.axis_index("c")
        sid = lax.axis_index("s")
        wid = sid * _NC + cid
        pltpu.sync_copy(zeros_hbm.at[pl.ds(sid * zero_rows, zero_rows)],
                        h_sh.at[pl.ds(sid * zero_rows, zero_rows)])
        plsc.subcore_barrier()

        pltpu.sync_copy(gidx_hbm.at[wid], gidx_v)
        pltpu.sync_copy(sidx_hbm.at[wid], sidx_v)

        def body(j, c):
            pltpu.sync_copy(table_hbm.at[gidx_v.at[j]], rows_v)
            pltpu.sync_copy(rows_v, h_sh.at[sidx_v.at[j]], add=True)
            return c

        n_chunks = jnp.where(cid == 0, _CPW0, _CPW1)
        lax.fori_loop(0, n_chunks, body, 0)

        plsc.subcore_barrier()
        pltpu.sync_copy(
            h_sh.at[pl.ds(sid * out_rows, out_rows)],
            out_hbm.at[cid, pl.ds(sid * out_rows, out_rows)])

    return k(table, gidx, sidx, zeros)


def kernel(x, edge_index, edge_type, emb_rel, weight_neighbor, a, W3):
    del a, W3
    N, D = x.shape
    R = emb_rel.shape[0]
    E = edge_type.shape[0]
    src = edge_index[0]
    dst = edge_index[1]

    T = ((N + R) // _BM + 1) * _BM
    xe = jnp.concatenate([x, emb_rel], axis=0)
    xe = jnp.pad(xe, ((0, T - (N + R)), (0, 0)))
    table = pl.pallas_call(
        _matmul_body,
        grid=(T // _BM,),
        in_specs=[
            pl.BlockSpec((_BM, D), lambda i: (i, 0)),
            pl.BlockSpec((D, D), lambda i: (0, 0)),
        ],
        out_specs=pl.BlockSpec((_BM, D), lambda i: (i, 0)),
        out_shape=jax.ShapeDtypeStruct((T, D), jnp.float32),
    )(xe, weight_neighbor)

    n_entries = 2 * E
    cpw_max = max(_CPW0, _CPW1)
    H = ((N + 1) // (_NS * 8) + 1) * (_NS * 8)
    capacity = _NS * (_CPW0 + _CPW1) * _CHUNK
    npad = capacity - n_entries
    gflat = jnp.concatenate([
        jnp.stack([src, N + edge_type], axis=1).reshape(-1),
        jnp.full((npad,), N + R, dtype=jnp.int32)])
    sflat = jnp.concatenate([
        jnp.stack([dst, dst], axis=1).reshape(-1),
        jnp.arange(npad, dtype=jnp.int32) % N])

    def to_worker_layout(flat):
        a3 = flat.reshape(_NS, _CPW0 + _CPW1, _CHUNK)
        c0 = a3[:, :_CPW0]
        c1 = jnp.pad(a3[:, _CPW0:], ((0, 0), (0, cpw_max - _CPW1), (0, 0)))
        both = jnp.stack(
            [jnp.pad(c0, ((0, 0), (0, cpw_max - _CPW0), (0, 0))), c1], axis=1)
        return both.reshape(_NS * _NC, cpw_max, _CHUNK)

    gidx = to_worker_layout(gflat)
    sidx = to_worker_layout(sflat)
    zeros = jnp.zeros((H, D), jnp.float32)

    partials = _sc_segment_sum(table, gidx, sidx, zeros, D, H)
    partials = partials[:, :N]

    bn = 1000
    spec = pl.BlockSpec((bn, D), lambda i: (i, 0))
    out = pl.pallas_call(
        _combine_body,
        grid=(N // bn,),
        in_specs=[spec, spec],
        out_specs=spec,
        out_shape=jax.ShapeDtypeStruct((N, D), jnp.float32),
    )(partials[0], partials[1])
    return out


# speedup vs baseline: 2.0364x; 2.0364x over previous
import functools

import jax
import jax.numpy as jnp
from jax import lax
from jax.experimental import pallas as pl
from jax.experimental.pallas import tpu as pltpu
from jax.experimental.pallas import tpu_sc as plsc

_NC = 2
_NS = 16
_NW = _NC * _NS
_CHUNK = 128
_BM = 512


def _matmul_body(xe_ref, w_ref, o_ref):
    o_ref[...] = jnp.dot(xe_ref[...], w_ref[...],
                         preferred_element_type=jnp.float32)


def _combine_body(p0_ref, p1_ref, o_ref):
    o_ref[...] = jnp.maximum(p0_ref[...] + p1_ref[...], 0.0)


def _sc_segment_sum(table, gx, ge, s_idx, zeros, N, D, H, RP, cpw):
    mesh = plsc.VectorSubcoreMesh(core_axis_name="c", subcore_axis_name="s")
    out_rows = H // _NS
    zero_rows = H // _NS

    @functools.partial(
        pl.kernel,
        out_type=jax.ShapeDtypeStruct((_NC, H, D), jnp.float32),
        mesh=mesh,
        scratch_types=[
            pltpu.VMEM((cpw, _CHUNK), jnp.int32),
            pltpu.VMEM((cpw, _CHUNK), jnp.int32),
            pltpu.VMEM((cpw, _CHUNK), jnp.int32),
            pltpu.VMEM((_CHUNK, D), jnp.float32),
            pltpu.VMEM_SHARED((RP, D), jnp.float32),
            pltpu.VMEM_SHARED((H, D), jnp.float32),
        ],
    )
    def k(table_hbm, gx_hbm, ge_hbm, s_hbm, zeros_hbm, out_hbm,
          gx_v, ge_v, s_v, rows_v, emb_sh, h_sh):
        cid = lax.axis_index("c")
        sid = lax.axis_index("s")
        wid = sid * _NC + cid
        pltpu.sync_copy(zeros_hbm,
                        h_sh.at[pl.ds(sid * zero_rows, zero_rows)])
        @pl.when(sid == 0)
        def _():
            pltpu.sync_copy(table_hbm.at[pl.ds(N, RP)], emb_sh)
        pltpu.sync_copy(gx_hbm.at[wid], gx_v)
        pltpu.sync_copy(ge_hbm.at[wid], ge_v)
        pltpu.sync_copy(s_hbm.at[wid], s_v)
        plsc.subcore_barrier()

        def body_x(j, c):
            pltpu.sync_copy(table_hbm.at[gx_v.at[j]], rows_v)
            pltpu.sync_copy(rows_v, h_sh.at[s_v.at[j]], add=True)
            return c

        def body_e(j, c):
            pltpu.sync_copy(emb_sh.at[ge_v.at[j]], rows_v)
            pltpu.sync_copy(rows_v, h_sh.at[s_v.at[j]], add=True)
            return c

        lax.fori_loop(0, cpw, body_x, 0)
        lax.fori_loop(0, cpw, body_e, 0)
        plsc.subcore_barrier()
        pltpu.sync_copy(
            h_sh.at[pl.ds(sid * out_rows, out_rows)],
            out_hbm.at[cid, pl.ds(sid * out_rows, out_rows)])

    return k(table, gx, ge, s_idx, zeros)


def _worker_chunks(vals, pad_val, cpw):
    per_w = vals.shape[0] // _NW
    v = vals.reshape(_NW, per_w)
    v = jnp.pad(v, ((0, 0), (0, cpw * _CHUNK - per_w)), constant_values=pad_val)
    return v.reshape(_NW, cpw, _CHUNK)


def kernel(x, edge_index, edge_type, emb_rel, weight_neighbor, a, W3):
    del a, W3
    N, D = x.shape
    R = emb_rel.shape[0]
    E = edge_type.shape[0]
    src = edge_index[0]
    dst = edge_index[1]

    T = ((N + R) // _BM + 1) * _BM
    xe = jnp.concatenate([x, emb_rel], axis=0)
    xe = jnp.pad(xe, ((0, T - (N + R)), (0, 0)))
    table = pl.pallas_call(
        _matmul_body,
        grid=(T // _BM,),
        in_specs=[
            pl.BlockSpec((_BM, D), lambda i: (i, 0)),
            pl.BlockSpec((D, D), lambda i: (0, 0)),
        ],
        out_specs=pl.BlockSpec((_BM, D), lambda i: (i, 0)),
        out_shape=jax.ShapeDtypeStruct((T, D), jnp.float32),
    )(xe, weight_neighbor)

    per_w = E // _NW
    cpw = -(-per_w // _CHUNK)
    H = ((N + 1) // (_NS * 8) + 1) * (_NS * 8)
    RP = ((R // 8) + 1) * 8
    n_pad_w = cpw * _CHUNK - per_w
    pad_s = (jnp.arange(_NW * n_pad_w, dtype=jnp.int32) % N).reshape(_NW, n_pad_w)
    gx = _worker_chunks(src, N + R, cpw)
    ge = _worker_chunks(edge_type, R, cpw)
    dst_w = dst.reshape(_NW, per_w)
    s_pad = jnp.concatenate([dst_w, pad_s], axis=1).reshape(_NW, cpw, _CHUNK)
    zeros = jnp.zeros((H // _NS, D), jnp.float32)

    partials = _sc_segment_sum(table, gx, ge, s_pad, zeros,
                               N, D, H, RP, cpw)
    partials = partials[:, :N]

    bn = 1000
    spec = pl.BlockSpec((bn, D), lambda i: (i, 0))
    out = pl.pallas_call(
        _combine_body,
        grid=(N // bn,),
        in_specs=[spec, spec],
        out_specs=spec,
        out_shape=jax.ShapeDtypeStruct((N, D), jnp.float32),
    )(partials[0], partials[1])
    return out


# speedup vs baseline: 2.0635x; 1.0133x over previous
import functools

import jax
import jax.numpy as jnp
from jax import lax
from jax.experimental import pallas as pl
from jax.experimental.pallas import tpu as pltpu
from jax.experimental.pallas import tpu_sc as plsc

_NC = 2
_NS = 16
_NW = _NC * _NS
_CHUNK = 128
_BM = 512


def _matmul_body(xe_ref, w_ref, o_ref):
    o_ref[...] = jnp.dot(xe_ref[...], w_ref[...],
                         preferred_element_type=jnp.float32)


def _combine_body(p0_ref, p1_ref, o_ref):
    o_ref[...] = jnp.maximum(p0_ref[0] + p1_ref[0], 0.0)


def _sc_segment_sum(table, gx, ge, s_idx, zeros, N, D, H, RP, cpw):
    mesh = plsc.VectorSubcoreMesh(core_axis_name="c", subcore_axis_name="s")
    out_rows = H // _NS
    zero_rows = H // _NS

    @functools.partial(
        pl.kernel,
        out_type=jax.ShapeDtypeStruct((_NC, H, D), jnp.float32),
        mesh=mesh,
        scratch_types=[
            pltpu.VMEM((cpw, _CHUNK), jnp.int32),
            pltpu.VMEM((cpw, _CHUNK), jnp.int32),
            pltpu.VMEM((cpw, _CHUNK), jnp.int32),
            pltpu.VMEM((_CHUNK, D), jnp.float32),
            pltpu.VMEM_SHARED((RP, D), jnp.float32),
            pltpu.VMEM_SHARED((H, D), jnp.float32),
        ],
    )
    def k(table_hbm, gx_hbm, ge_hbm, s_hbm, zeros_hbm, out_hbm,
          gx_v, ge_v, s_v, rows_v, emb_sh, h_sh):
        cid = lax.axis_index("c")
        sid = lax.axis_index("s")
        wid = sid * _NC + cid
        pltpu.sync_copy(zeros_hbm,
                        h_sh.at[pl.ds(sid * zero_rows, zero_rows)])
        @pl.when(sid == 0)
        def _():
            pltpu.sync_copy(table_hbm.at[pl.ds(N, RP)], emb_sh)
        pltpu.sync_copy(gx_hbm.at[wid], gx_v)
        pltpu.sync_copy(ge_hbm.at[wid], ge_v)
        pltpu.sync_copy(s_hbm.at[wid], s_v)
        plsc.subcore_barrier()

        def body_x(j, c):
            pltpu.sync_copy(table_hbm.at[gx_v.at[j]], rows_v)
            pltpu.sync_copy(rows_v, h_sh.at[s_v.at[j]], add=True)
            return c

        def body_e(j, c):
            pltpu.sync_copy(emb_sh.at[ge_v.at[j]], rows_v)
            pltpu.sync_copy(rows_v, h_sh.at[s_v.at[j]], add=True)
            return c

        lax.fori_loop(0, cpw, body_x, 0)
        lax.fori_loop(0, cpw, body_e, 0)
        plsc.subcore_barrier()
        pltpu.sync_copy(
            h_sh.at[pl.ds(sid * out_rows, out_rows)],
            out_hbm.at[cid, pl.ds(sid * out_rows, out_rows)])

    return k(table, gx, ge, s_idx, zeros)


def _worker_chunks(vals, pad_val, cpw):
    per_w = vals.shape[0] // _NW
    v = vals.reshape(_NW, per_w)
    v = jnp.pad(v, ((0, 0), (0, cpw * _CHUNK - per_w)), constant_values=pad_val)
    return v.reshape(_NW, cpw, _CHUNK)


def kernel(x, edge_index, edge_type, emb_rel, weight_neighbor, a, W3):
    del a, W3
    N, D = x.shape
    R = emb_rel.shape[0]
    E = edge_type.shape[0]
    src = edge_index[0]
    dst = edge_index[1]

    T = ((N + R) // _BM + 1) * _BM
    xe = jnp.concatenate([x, emb_rel], axis=0)
    xe = jnp.pad(xe, ((0, T - (N + R)), (0, 0)))
    table = pl.pallas_call(
        _matmul_body,
        grid=(T // _BM,),
        in_specs=[
            pl.BlockSpec((_BM, D), lambda i: (i, 0)),
            pl.BlockSpec((D, D), lambda i: (0, 0)),
        ],
        out_specs=pl.BlockSpec((_BM, D), lambda i: (i, 0)),
        out_shape=jax.ShapeDtypeStruct((T, D), jnp.float32),
    )(xe, weight_neighbor)

    per_w = E // _NW
    cpw = -(-per_w // _CHUNK)
    H = ((N + 1) // (_NS * 8) + 1) * (_NS * 8)
    RP = ((R // 8) + 1) * 8
    n_pad_w = cpw * _CHUNK - per_w
    pad_s = (jnp.arange(_NW * n_pad_w, dtype=jnp.int32) % N).reshape(_NW, n_pad_w)
    gx = _worker_chunks(src, N + R, cpw)
    ge = _worker_chunks(edge_type, R, cpw)
    dst_w = dst.reshape(_NW, per_w)
    s_pad = jnp.concatenate([dst_w, pad_s], axis=1).reshape(_NW, cpw, _CHUNK)
    zeros = jnp.zeros((H // _NS, D), jnp.float32)

    partials = _sc_segment_sum(table, gx, ge, s_pad, zeros,
                               N, D, H, RP, cpw)

    bn = 1000
    out = pl.pallas_call(
        _combine_body,
        grid=(N // bn,),
        in_specs=[
            pl.BlockSpec((1, bn, D), lambda i: (0, i, 0)),
            pl.BlockSpec((1, bn, D), lambda i: (1, i, 0)),
        ],
        out_specs=pl.BlockSpec((bn, D), lambda i: (i, 0)),
        out_shape=jax.ShapeDtypeStruct((N, D), jnp.float32),
    )(partials, partials)
    return out
